# baseline (speedup 1.0000x reference)
import functools

import jax
import jax.numpy as jnp
from jax import lax
from jax.experimental import pallas as pl
from jax.experimental.pallas import tpu as pltpu
from jax.experimental.pallas import tpu_sc as plsc

B = 16384
H = 200
D = 64
NC = 2
NS = 16
NW = NC * NS
BPW = B // NW
CH = 128
NCHUNK = BPW // CH
LANES = 16


def _body(hist_hbm, seqlen_hbm, uid_hbm, utab_hbm, itab_hbm, gemb_hbm,
          out_hbm,
          seq_v, uid_v, hidx_v, lastid_v, urows_v, irows_v, gemb_v,
          sem_u, sem_i, sem_h):
    wid = lax.axis_index("s") * NC + lax.axis_index("c")
    base = wid * BPW

    pltpu.sync_copy(seqlen_hbm.at[pl.ds(base, BPW)], seq_v)
    for j in range(NCHUNK):
        pltpu.sync_copy(uid_hbm.at[pl.ds(base + j * CH, CH)], uid_v.at[j])
    pltpu.sync_copy(gemb_hbm, gemb_v)

    ucps = [pltpu.async_copy(utab_hbm.at[uid_v.at[j]],
                             urows_v.at[pl.ds(j * CH, CH)], sem_u)
            for j in range(NCHUNK)]

    lane = lax.iota(jnp.int32, LANES)
    for j in range(NCHUNK):
        for i in range(CH // LANES):
            off = j * CH + i * LANES
            rows = base + off + lane
            sl = seq_v[pl.ds(off, LANES)]
            hidx_v[j, pl.ds(i * LANES, LANES)] = rows * H + sl - 1

    hcps = [pltpu.async_copy(hist_hbm.at[hidx_v.at[j]], lastid_v.at[j], sem_h)
            for j in range(NCHUNK)]
    for c in hcps:
        c.wait()
    icps = [pltpu.async_copy(itab_hbm.at[lastid_v.at[j]],
                             irows_v.at[pl.ds(j * CH, CH)], sem_i)
            for j in range(NCHUNK)]
    for c in ucps:
        c.wait()
    for c in icps:
        c.wait()

    g = [gemb_v[pl.ds(k * LANES, LANES)] for k in range(D // LANES)]

    def add_row(i, _):
        for k in range(D // LANES):
            s = pl.ds(k * LANES, LANES)
            urows_v[i, s] = urows_v[i, s] + irows_v[i, s] + g[k]
        return 0

    lax.fori_loop(0, BPW, add_row, 0)

    pltpu.sync_copy(urows_v, out_hbm.at[pl.ds(base, BPW)])


@functools.partial(jax.jit, static_argnames=())
def kernel(in_item_id, seqlen, user_id, user_table, item_table,
           global_user_emb):
    hist_flat = in_item_id.reshape(-1)
    run = pl.kernel(
        _body,
        out_type=jax.ShapeDtypeStruct((B, D), jnp.float32),
        mesh=plsc.VectorSubcoreMesh(core_axis_name="c", subcore_axis_name="s"),
        compiler_params=pltpu.CompilerParams(use_tc_tiling_on_sc=False),
        scratch_types=[
            pltpu.VMEM((BPW,), jnp.int32),
            pltpu.VMEM((NCHUNK, CH), jnp.int32),
            pltpu.VMEM((NCHUNK, CH), jnp.int32),
            pltpu.VMEM((NCHUNK, CH), jnp.int32),
            pltpu.VMEM((BPW, D), jnp.float32),
            pltpu.VMEM((BPW, D), jnp.float32),
            pltpu.VMEM((D,), jnp.float32),
            pltpu.SemaphoreType.DMA,
            pltpu.SemaphoreType.DMA,
            pltpu.SemaphoreType.DMA,
        ],
    )
    return run(hist_flat, seqlen, user_id, user_table, item_table,
               global_user_emb)


# speedup vs baseline: 2.2413x; 2.2413x over previous
import jax
import jax.numpy as jnp
from jax import lax
from jax.experimental import pallas as pl
from jax.experimental.pallas import tpu as pltpu
from jax.experimental.pallas import tpu_sc as plsc

B = 16384
H = 200
NUSERS = 1000000
D = 64
NC = 2
NS = 16
NW = NC * NS
BPW = B // NW
LANES = 16
HCH = 128


def _extract_scalar(vec, lane, l):
    return jnp.max(jnp.where(lane == l, vec, 0))


def _body(hist_hbm, seqlen_hbm, uid_hbm, ut_hbm, it_hbm, gemb_hbm,
          out_hbm,
          seq_v, uid_v, lastid_v, hist_v, gemb_v, out_v,
          ub0, ub1, ib0, ib1, su0, su1, si0, si1):
    wid = lax.axis_index("s") * NC + lax.axis_index("c")
    base = wid * BPW
    lane = lax.iota(jnp.int32, LANES)

    pltpu.sync_copy(seqlen_hbm.at[pl.ds(base, BPW)], seq_v)
    pltpu.sync_copy(uid_hbm.at[pl.ds(base, BPW)], uid_v)
    pltpu.sync_copy(gemb_hbm, gemb_v)
    gvec = [gemb_v[pl.ds(k * LANES, LANES)] for k in range(D // LANES)]

    for blk in range(BPW // HCH):
        pltpu.sync_copy(hist_hbm.at[:, pl.ds(base + blk * HCH, HCH)], hist_v)
        for grp in range(HCH // LANES):
            off = blk * HCH + grp * LANES
            rows = seq_v[pl.ds(off, LANES)] - 1
            cols = grp * LANES + lane
            lastid_v[pl.ds(off, LANES)] = plsc.load_gather(
                hist_v, [rows, cols])

    ubufs = [ub0, ub1]
    ibufs = [ib0, ib1]
    usems = [su0, su1]
    isems = [si0, si1]

    def _fetch(r, s):
        g16 = lax.shift_right_logical(r, 4)
        voff = pl.multiple_of(g16 * LANES, LANES)
        l = r - g16 * LANES
        ru = _extract_scalar(uid_v[pl.ds(voff, LANES)], lane, l)
        ri = _extract_scalar(lastid_v[pl.ds(voff, LANES)], lane, l)
        ju = pl.multiple_of(lax.shift_right_logical(ru, 7) * 128, 128)
        ji = pl.multiple_of(lax.shift_right_logical(ri, 7) * 128, 128)
        pltpu.async_copy(ut_hbm.at[:, pl.ds(ju, 128)], ubufs[s], usems[s])
        pltpu.async_copy(it_hbm.at[:, pl.ds(ji, 128)], ibufs[s], isems[s])
        return ru, ri

    def _consume(r, s, ru, ri):
        cu = jnp.broadcast_to(ru & 127, (LANES,))
        ci = jnp.broadcast_to(ri & 127, (LANES,))
        pltpu.make_async_copy(
            ut_hbm.at[:, pl.ds(0, 128)], ubufs[s], usems[s]).wait()
        pltpu.make_async_copy(
            it_hbm.at[:, pl.ds(0, 128)], ibufs[s], isems[s]).wait()
        for k in range(D // LANES):
            dvec = k * LANES + lane
            uval = plsc.load_gather(ubufs[s], [dvec, cu])
            ival = plsc.load_gather(ibufs[s], [dvec, ci])
            out_v[r, pl.ds(k * LANES, LANES)] = uval + ival + gvec[k]

    r0u, r0i = _fetch(jnp.int32(0), 0)
    r1u, r1i = _fetch(jnp.int32(1), 1)

    def step(it, carry):
        au, ai, bu, bi = carry
        r = 2 * it
        _consume(r, 0, au, ai)
        nu, ni = _fetch(lax.rem(r + 2, BPW), 0)
        _consume(r + 1, 1, bu, bi)
        mu, mi = _fetch(lax.rem(r + 3, BPW), 1)
        return nu, ni, mu, mi

    lax.fori_loop(0, BPW // 2, step, (r0u, r0i, r1u, r1i))

    for s in range(2):
        pltpu.make_async_copy(
            ut_hbm.at[:, pl.ds(0, 128)], ubufs[s], usems[s]).wait()
        pltpu.make_async_copy(
            it_hbm.at[:, pl.ds(0, 128)], ibufs[s], isems[s]).wait()

    pltpu.sync_copy(out_v, out_hbm.at[pl.ds(base, BPW)])


@jax.jit
def kernel(in_item_id, seqlen, user_id, user_table, item_table,
           global_user_emb):
    ut = user_table.T
    it = item_table.T
    hist = in_item_id.T
    run = pl.kernel(
        _body,
        out_type=jax.ShapeDtypeStruct((B, D), jnp.float32),
        mesh=plsc.VectorSubcoreMesh(core_axis_name="c", subcore_axis_name="s"),
        compiler_params=pltpu.CompilerParams(
            use_tc_tiling_on_sc=True, disable_bounds_checks=True,
            needs_layout_passes=False),
        scratch_types=[
            pltpu.VMEM((BPW,), jnp.int32),
            pltpu.VMEM((BPW,), jnp.int32),
            pltpu.VMEM((BPW,), jnp.int32),
            pltpu.VMEM((H, HCH), jnp.int32),
            pltpu.VMEM((D,), jnp.float32),
            pltpu.VMEM((BPW, D), jnp.float32),
            pltpu.VMEM((D, 128), jnp.float32),
            pltpu.VMEM((D, 128), jnp.float32),
            pltpu.VMEM((D, 128), jnp.float32),
            pltpu.VMEM((D, 128), jnp.float32),
            pltpu.SemaphoreType.DMA,
            pltpu.SemaphoreType.DMA,
            pltpu.SemaphoreType.DMA,
            pltpu.SemaphoreType.DMA,
        ],
    )
    return run(hist, seqlen, user_id, ut, it, global_user_emb)


# speedup vs baseline: 2.7130x; 1.2104x over previous
import jax
import jax.numpy as jnp
from jax import lax
from jax.experimental import pallas as pl
from jax.experimental.pallas import tpu as pltpu
from jax.experimental.pallas import tpu_sc as plsc

B = 16384
H = 200
NUSERS = 1000000
D = 64
NC = 2
NS = 16
NW = NC * NS
BPW = B // NW
LANES = 16
HCH = 128
NSLOT = 4


def _extract_scalar(vec, lane, l):
    return jnp.max(jnp.where(lane == l, vec, 0))


def _body(hist_hbm, seqlen_hbm, uid_hbm, ut_hbm, it_hbm, gemb_hbm,
          out_hbm,
          seq_v, uid_v, lastid_v, hist_v, gemb_v, out_v,
          *bufsems):
    ubufs = bufsems[:NSLOT]
    ibufs = bufsems[NSLOT:2 * NSLOT]
    usems = bufsems[2 * NSLOT:3 * NSLOT]
    isems = bufsems[3 * NSLOT:4 * NSLOT]

    wid = lax.axis_index("s") * NC + lax.axis_index("c")
    base = wid * BPW
    lane = lax.iota(jnp.int32, LANES)

    pltpu.sync_copy(seqlen_hbm.at[pl.ds(base, BPW)], seq_v)
    pltpu.sync_copy(uid_hbm.at[pl.ds(base, BPW)], uid_v)
    pltpu.sync_copy(gemb_hbm, gemb_v)
    gvec = [gemb_v[pl.ds(k * LANES, LANES)] for k in range(D // LANES)]

    for blk in range(BPW // HCH):
        pltpu.sync_copy(hist_hbm.at[:, pl.ds(base + blk * HCH, HCH)], hist_v)
        for grp in range(HCH // LANES):
            off = blk * HCH + grp * LANES
            rows = seq_v[pl.ds(off, LANES)] - 1
            cols = grp * LANES + lane
            lastid_v[pl.ds(off, LANES)] = plsc.load_gather(
                hist_v, [rows, cols])

    def _fetch(r, s):
        g16 = lax.shift_right_logical(r, 4)
        voff = pl.multiple_of(g16 * LANES, LANES)
        l = r - g16 * LANES
        ru = _extract_scalar(uid_v[pl.ds(voff, LANES)], lane, l)
        ri = _extract_scalar(lastid_v[pl.ds(voff, LANES)], lane, l)
        ju = pl.multiple_of(lax.shift_right_logical(ru, 7) * 128, 128)
        ji = pl.multiple_of(lax.shift_right_logical(ri, 7) * 128, 128)
        pltpu.async_copy(ut_hbm.at[:, pl.ds(ju, 128)], ubufs[s], usems[s])
        pltpu.async_copy(it_hbm.at[:, pl.ds(ji, 128)], ibufs[s], isems[s])
        return ru, ri

    def _consume(r, hbase, s, ru, ri):
        cu = jnp.broadcast_to(ru & 127, (LANES,))
        ci = jnp.broadcast_to(ri & 127, (LANES,))
        pltpu.make_async_copy(
            ut_hbm.at[:, pl.ds(0, 128)], ubufs[s], usems[s]).wait()
        pltpu.make_async_copy(
            it_hbm.at[:, pl.ds(0, 128)], ibufs[s], isems[s]).wait()
        for k in range(D // LANES):
            dvec = k * LANES + lane
            uval = plsc.load_gather(ubufs[s], [dvec, cu])
            ival = plsc.load_gather(ibufs[s], [dvec, ci])
            out_v[r - hbase, pl.ds(k * LANES, LANES)] = uval + ival + gvec[k]

    HB = BPW // 2
    for half in range(2):
        hbase = half * HB
        carry0 = []
        for s in range(NSLOT):
            carry0.extend(_fetch(jnp.int32(hbase + s), s))

        def step(it, carry, hbase=hbase):
            rbase = hbase + NSLOT * it
            out = []
            for s in range(NSLOT):
                _consume(rbase + s, hbase, s, carry[2 * s], carry[2 * s + 1])
                nxt = hbase + lax.rem(rbase + s + NSLOT - hbase, HB)
                out.extend(_fetch(nxt, s))
            return tuple(out)

        lax.fori_loop(0, HB // NSLOT, step, tuple(carry0))

        for s in range(NSLOT):
            pltpu.make_async_copy(
                ut_hbm.at[:, pl.ds(0, 128)], ubufs[s], usems[s]).wait()
            pltpu.make_async_copy(
                it_hbm.at[:, pl.ds(0, 128)], ibufs[s], isems[s]).wait()

        pltpu.sync_copy(out_v, out_hbm.at[pl.ds(base + hbase, HB)])


@jax.jit
def kernel(in_item_id, seqlen, user_id, user_table, item_table,
           global_user_emb):
    ut = user_table.T
    it = item_table.T
    hist = in_item_id.T
    run = pl.kernel(
        _body,
        out_type=jax.ShapeDtypeStruct((B, D), jnp.float32),
        mesh=plsc.VectorSubcoreMesh(core_axis_name="c", subcore_axis_name="s"),
        compiler_params=pltpu.CompilerParams(
            use_tc_tiling_on_sc=True, disable_bounds_checks=True,
            needs_layout_passes=False),
        scratch_types=[
            pltpu.VMEM((BPW,), jnp.int32),
            pltpu.VMEM((BPW,), jnp.int32),
            pltpu.VMEM((BPW,), jnp.int32),
            pltpu.VMEM((H, HCH), jnp.int32),
            pltpu.VMEM((D,), jnp.float32),
            pltpu.VMEM((BPW // 2, D), jnp.float32),
        ] + [pltpu.VMEM((D, 128), jnp.float32)] * (2 * NSLOT)
          + [pltpu.SemaphoreType.DMA] * (2 * NSLOT),
    )
    return run(hist, seqlen, user_id, ut, it, global_user_emb)
